# baseline (speedup 1.0000x reference)
import functools
import jax
import jax.numpy as jnp
from jax import lax
from jax.experimental import pallas as pl
from jax.experimental.pallas import tpu as pltpu
from jax.experimental.pallas import tpu_sc as plsc

NI = 64
CLEAN = 48
D = 256
GP = 64

SC_CORES = 2
SC_SUBCORES = 16
NW = SC_CORES * SC_SUBCORES


def _pair_idx(S0, S1):
    dn = (((1,), (1,)), ((), ()))
    sim0 = lax.dot_general(S0, S0, dn, preferred_element_type=jnp.float32)
    sim1 = lax.dot_general(S1, S1, dn, preferred_element_type=jnp.float32)
    SIM2 = jnp.concatenate([sim0, sim1], axis=1)

    H = NI // 2
    SL = SIM2[:H, :]
    SH = SIM2[H:, :]
    ko = lax.broadcasted_iota(jnp.int32, (H, H, 2 * NI), 0)
    jo = lax.broadcasted_iota(jnp.int32, (H, H, 2 * NI), 1)
    tie = ko < jo
    def diag_block(S):
        a = S[None, :, :]
        b = S[:, None, :]
        c = ((b < a) | ((b == a) & tie)).astype(jnp.float32)
        cf = c.reshape(4, H // 4, H, 2 * NI)
        return jnp.sum(jnp.sum(cf, axis=1), axis=0)
    R_LL = diag_block(SL)
    R_HH = diag_block(SH)
    c_LH = (SL[:, None, :] <= SH[None, :, :]).astype(jnp.float32)
    cf_LH = c_LH.reshape(4, H // 4, H, 2 * NI)
    R_LH = jnp.sum(jnp.sum(cf_LH, axis=1), axis=0)
    Q = jnp.sum(c_LH, axis=1)
    R_L = R_LL + (float(H) - Q)
    R_H = R_HH + R_LH
    R = jnp.concatenate([R_L, R_H], axis=0)

    gi = lax.broadcasted_iota(jnp.int32, (2 * NI, 2), 0)
    gc = lax.broadcasted_iota(jnp.int32, (2 * NI, 2), 1)
    SEL = ((gi // NI) == gc).astype(jnp.float32)
    colsum = lax.dot_general(R, SEL, (((1,), (0,)), ((), ())),
                             preferred_element_type=jnp.float32)
    colsumT = lax.dot_general(SEL, R, (((0,), (1,)), ((), ())),
                              preferred_element_type=jnp.float32)

    j_col = lax.broadcasted_iota(jnp.int32, (NI, 2), 0).astype(jnp.float32)
    key = colsum * 64.0 + (63.0 - j_col)
    j_rowT = lax.broadcasted_iota(jnp.int32, (2, NI), 1).astype(jnp.float32)
    keyT = colsumT * 64.0 + (63.0 - j_rowT)

    p_row = lax.broadcasted_iota(jnp.int32, (NI, NI), 1).astype(jnp.float32)
    jj = lax.broadcasted_iota(jnp.int32, (NI, NI), 0).astype(jnp.float32)

    idxs = []
    for g in (0, 1):
        key_col = key[:, g:g + 1]
        key_row = keyT[g:g + 1, :]
        M = (key_row > key_col).astype(jnp.float32)
        pos = jnp.sum(M, axis=1, keepdims=True)
        OT = (pos == p_row).astype(jnp.float32)
        idxs.append(jnp.sum(OT * jj, axis=0, keepdims=True))
    return jnp.concatenate(idxs, axis=0).astype(jnp.int32)


def _tc_body(x_ref, glob_ref, loc_ref, goff=0):
    pid = pl.program_id(0) + goff
    for p in range(GP // 2):
        S0 = x_ref[(2 * p) * NI:(2 * p + 1) * NI, :]
        S1 = x_ref[(2 * p + 1) * NI:(2 * p + 2) * NI, :]
        idx = _pair_idx(S0, S1)[:, :CLEAN]
        base = (pid * GP + 2 * p) * NI
        gof = lax.broadcasted_iota(jnp.int32, (2, CLEAN), 0) * NI
        loc_ref[0, 2 * p:2 * p + 2, :] = idx
        glob_ref[0, 2 * p:2 * p + 2, :] = idx + gof + base


def _tc_indices(x, num_split, goff):
    steps = num_split // GP
    glob3, loc3 = pl.pallas_call(
        functools.partial(_tc_body, goff=goff),
        grid=(steps,),
        in_specs=[pl.BlockSpec((GP * NI, D), lambda g: (g + goff, 0))],
        out_specs=[pl.BlockSpec((1, GP, CLEAN), lambda g: (g, 0, 0)),
                   pl.BlockSpec((1, GP, CLEAN), lambda g: (g, 0, 0))],
        out_shape=[jax.ShapeDtypeStruct((steps, GP, CLEAN), jnp.int32),
                   jax.ShapeDtypeStruct((steps, GP, CLEAN), jnp.int32)],
    )(x)
    return glob3.reshape(-1), loc3.reshape(num_split, CLEAN)


def _sc_gather(x, flat_idx, n_rows):
    bpw = n_rows // NW
    mesh = plsc.VectorSubcoreMesh(core_axis_name="c", subcore_axis_name="s")

    @functools.partial(
        pl.kernel, mesh=mesh,
        out_type=jax.ShapeDtypeStruct((n_rows, D), jnp.float32),
        scratch_types=[
            pltpu.VMEM((bpw,), jnp.int32),
            pltpu.VMEM((bpw, D), jnp.float32),
            pltpu.SemaphoreType.DMA,
        ],
    )
    def k(x_hbm, idx_hbm, out_hbm, idx_v, rows_v, sem):
        wid = lax.axis_index("s") * SC_CORES + lax.axis_index("c")
        base = wid * bpw
        pltpu.sync_copy(idx_hbm.at[pl.ds(base, bpw)], idx_v)
        pltpu.async_copy(x_hbm.at[idx_v], rows_v, sem).wait()
        pltpu.sync_copy(rows_v, out_hbm.at[pl.ds(base, bpw)])

    return k(x, flat_idx)


@jax.jit
def kernel(x):
    B = x.shape[0]
    num_split = B // NI
    half = num_split // 2
    fi0, ci0 = _tc_indices(x, half, 0)
    fi1, ci1 = _tc_indices(x, half, half // GP)
    cd0 = _sc_gather(x, fi0, half * CLEAN)
    cd1 = _sc_gather(x, fi1, half * CLEAN)
    clean_data = jnp.concatenate([cd0, cd1], axis=0)
    clean_indices = jnp.concatenate([ci0, ci1], axis=0)
    return (clean_data, clean_indices)


# speedup vs baseline: 1.1168x; 1.1168x over previous
import functools
import jax
import jax.numpy as jnp
from jax import lax
from jax.experimental import pallas as pl
from jax.experimental.pallas import tpu as pltpu
from jax.experimental.pallas import tpu_sc as plsc

NI = 64
CLEAN = 48
D = 256
GP = 64

SC_CORES = 2
SC_SUBCORES = 16
NW = SC_CORES * SC_SUBCORES


def _pair_idx(S0, S1):
    dn = (((1,), (1,)), ((), ()))
    sim0 = lax.dot_general(S0, S0, dn, preferred_element_type=jnp.float32)
    sim1 = lax.dot_general(S1, S1, dn, preferred_element_type=jnp.float32)
    SIM2 = jnp.concatenate([sim0, sim1], axis=1)

    H = NI // 2
    SL = SIM2[:H, :]
    SH = SIM2[H:, :]
    ko = lax.broadcasted_iota(jnp.int32, (H, H, 2 * NI), 0)
    jo = lax.broadcasted_iota(jnp.int32, (H, H, 2 * NI), 1)
    tie = ko < jo
    def diag_block(S):
        a = S[None, :, :]
        b = S[:, None, :]
        c = ((b < a) | ((b == a) & tie)).astype(jnp.float32)
        cf = c.reshape(4, H // 4, H, 2 * NI)
        return jnp.sum(jnp.sum(cf, axis=1), axis=0)
    R_LL = diag_block(SL)
    R_HH = diag_block(SH)
    c_LH = (SL[:, None, :] <= SH[None, :, :]).astype(jnp.float32)
    cf_LH = c_LH.reshape(4, H // 4, H, 2 * NI)
    R_LH = jnp.sum(jnp.sum(cf_LH, axis=1), axis=0)
    Q = jnp.sum(c_LH, axis=1)
    R_L = R_LL + (float(H) - Q)
    R_H = R_HH + R_LH
    R = jnp.concatenate([R_L, R_H], axis=0)

    gi = lax.broadcasted_iota(jnp.int32, (2 * NI, 2), 0)
    gc = lax.broadcasted_iota(jnp.int32, (2 * NI, 2), 1)
    SEL = ((gi // NI) == gc).astype(jnp.float32)
    colsum = lax.dot_general(R, SEL, (((1,), (0,)), ((), ())),
                             preferred_element_type=jnp.float32)
    colsumT = lax.dot_general(SEL, R, (((0,), (1,)), ((), ())),
                              preferred_element_type=jnp.float32)

    j_col = lax.broadcasted_iota(jnp.int32, (NI, 2), 0).astype(jnp.float32)
    key = colsum * 64.0 + (63.0 - j_col)
    j_rowT = lax.broadcasted_iota(jnp.int32, (2, NI), 1).astype(jnp.float32)
    keyT = colsumT * 64.0 + (63.0 - j_rowT)

    p_row = lax.broadcasted_iota(jnp.int32, (NI, NI), 1).astype(jnp.float32)
    jj = lax.broadcasted_iota(jnp.int32, (NI, NI), 0).astype(jnp.float32)

    idxs = []
    for g in (0, 1):
        key_col = key[:, g:g + 1]
        key_row = keyT[g:g + 1, :]
        M = (key_row > key_col).astype(jnp.float32)
        pos = jnp.sum(M, axis=1, keepdims=True)
        OT = (pos == p_row).astype(jnp.float32)
        idxs.append(jnp.sum(OT * jj, axis=0, keepdims=True))
    return jnp.concatenate(idxs, axis=0).astype(jnp.int32)


def _tc_body(x_ref, glob_ref, loc_ref):
    pid = pl.program_id(0)
    for p in range(GP // 2):
        S0 = x_ref[(2 * p) * NI:(2 * p + 1) * NI, :]
        S1 = x_ref[(2 * p + 1) * NI:(2 * p + 2) * NI, :]
        idx = _pair_idx(S0, S1)[:, :CLEAN]
        base = (pid * GP + 2 * p) * NI
        gof = lax.broadcasted_iota(jnp.int32, (2, CLEAN), 0) * NI
        loc_ref[0, 2 * p:2 * p + 2, :] = idx
        glob_ref[0, 2 * p:2 * p + 2, :] = idx + gof + base


def _tc_indices(x, num_split):
    steps = num_split // GP
    glob3, loc3 = pl.pallas_call(
        _tc_body,
        grid=(steps,),
        in_specs=[pl.BlockSpec((GP * NI, D), lambda g: (g, 0))],
        out_specs=[pl.BlockSpec((1, GP, CLEAN), lambda g: (g, 0, 0)),
                   pl.BlockSpec((1, GP, CLEAN), lambda g: (g, 0, 0))],
        out_shape=[jax.ShapeDtypeStruct((steps, GP, CLEAN), jnp.int32),
                   jax.ShapeDtypeStruct((steps, GP, CLEAN), jnp.int32)],
    )(x)
    return glob3.reshape(-1), loc3.reshape(num_split, CLEAN)


def _sc_gather(x, flat_idx, n_rows):
    bpw = n_rows // NW
    mesh = plsc.VectorSubcoreMesh(core_axis_name="c", subcore_axis_name="s")

    @functools.partial(
        pl.kernel, mesh=mesh,
        out_type=jax.ShapeDtypeStruct((n_rows, D), jnp.float32),
        scratch_types=[
            pltpu.VMEM((bpw,), jnp.int32),
            pltpu.VMEM((bpw, D), jnp.float32),
            pltpu.SemaphoreType.DMA,
        ],
    )
    def k(x_hbm, idx_hbm, out_hbm, idx_v, rows_v, sem):
        wid = lax.axis_index("s") * SC_CORES + lax.axis_index("c")
        base = wid * bpw
        pltpu.sync_copy(idx_hbm.at[pl.ds(base, bpw)], idx_v)
        pltpu.async_copy(x_hbm.at[idx_v], rows_v, sem).wait()
        pltpu.sync_copy(rows_v, out_hbm.at[pl.ds(base, bpw)])

    return k(x, flat_idx)


@jax.jit
def kernel(x):
    B = x.shape[0]
    num_split = B // NI
    flat_idx, clean_indices = _tc_indices(x, num_split)
    clean_data = _sc_gather(x, flat_idx, num_split * CLEAN)
    return (clean_data, clean_indices)
